# baseline (speedup 1.0000x reference)
import functools

import jax
import jax.numpy as jnp
from jax import lax
from jax.experimental import pallas as pl
from jax.experimental.pallas import tpu as pltpu
from jax.experimental.pallas import tpu_sc as plsc

NC = 2
NS = 16
NW = NC * NS
LANES = 16
K = 112
NSLOT = 3

F32 = jnp.float32
I32 = jnp.int32


def _mesh():
    return plsc.VectorSubcoreMesh(core_axis_name="c", subcore_axis_name="s")


def _sc_degrees(P, EPW):
    @functools.partial(
        pl.kernel,
        out_type=jax.ShapeDtypeStruct((NW, P), F32),
        mesh=_mesh(),
        compiler_params=pltpu.CompilerParams(needs_layout_passes=False),
        scratch_types=[
            pltpu.VMEM((EPW,), I32),
            pltpu.VMEM((P,), F32),
        ],
    )
    def k(dst_hbm, out_hbm, didx_v, acc_v):
        c = lax.axis_index("c")
        s = lax.axis_index("s")
        w = c * NS + s

        def zero(i, _):
            acc_v[pl.ds(i * LANES, LANES)] = jnp.zeros((LANES,), F32)
            return 0

        lax.fori_loop(0, P // LANES, zero, 0)
        pltpu.sync_copy(dst_hbm.at[w], didx_v)
        ones16 = jnp.ones((LANES,), F32)

        def body(j, _):
            idx = didx_v[pl.ds(j * LANES, LANES)]
            plsc.addupdate_scatter(acc_v, [idx], ones16)
            return 0

        lax.fori_loop(0, EPW // LANES, body, 0)
        pltpu.sync_copy(acc_v, out_hbm.at[w])

    return k


def _sc_rows(P, NCHUNK):
    STRIPE = P // NS

    @functools.partial(
        pl.kernel,
        out_type=jax.ShapeDtypeStruct((NC, P, 128), F32),
        mesh=_mesh(),
        compiler_params=pltpu.CompilerParams(needs_layout_passes=False),
        scratch_types=(
            [pltpu.VMEM((K,), I32) for _ in range(2 * NSLOT)]
            + [pltpu.VMEM((K,), I32) for _ in range(2 * NSLOT)]
            + [pltpu.VMEM((K, 128), F32) for _ in range(NSLOT)]
            + [pltpu.VMEM_SHARED((P, 128), F32)]
            + [pltpu.SemaphoreType.DMA] * (3 * NSLOT)
        ),
    )
    def k(g_hbm, src_hbm, dst_hbm, out_hbm, *refs):
        sidx = refs[0:2 * NSLOT]
        didx = refs[2 * NSLOT:4 * NSLOT]
        rows = refs[4 * NSLOT:4 * NSLOT + NSLOT]
        acc_sh = refs[4 * NSLOT + NSLOT]
        sem_i = refs[4 * NSLOT + NSLOT + 1:4 * NSLOT + NSLOT + 1 + NSLOT]
        sem_g = refs[4 * NSLOT + NSLOT + 1 + NSLOT:4 * NSLOT + NSLOT + 1 + 2 * NSLOT]
        sem_s = refs[4 * NSLOT + NSLOT + 1 + 2 * NSLOT:]

        c = lax.axis_index("c")
        s = lax.axis_index("s")
        w = c * NS + s

        zero16 = jnp.zeros((LANES,), F32)

        def zrow(r, _):
            for j in range(128 // LANES):
                rows[0][r, pl.ds(j * LANES, LANES)] = zero16
            return 0

        lax.fori_loop(0, K, zrow, 0)
        done = 0
        while done < STRIPE:
            n = min(K, STRIPE - done)
            pltpu.sync_copy(rows[0].at[pl.ds(0, n)],
                            acc_sh.at[pl.ds(s * STRIPE + done, n)])
            done += n
        plsc.subcore_barrier()

        def idx_load(ci, ring, sem):
            pltpu.async_copy(src_hbm.at[w, ci], sidx[ring], sem)
            pltpu.async_copy(dst_hbm.at[w, ci], didx[ring], sem)

        def idx_wait(ci, ring, sem):
            pltpu.make_async_copy(src_hbm.at[w, ci], sidx[ring], sem).wait()
            pltpu.make_async_copy(dst_hbm.at[w, ci], didx[ring], sem).wait()

        for j in range(NSLOT):
            idx_load(j, j, sem_i[j])

        def round_(k2, _):
            c0 = 2 * NSLOT * k2
            for half in range(2):
                for j in range(NSLOT):
                    m = half * NSLOT + j
                    ci = c0 + m
                    idx_wait(ci, m, sem_i[j])
                    prev = (m + NSLOT) % (2 * NSLOT)
                    drain = lambda j=j, prev=prev: pltpu.make_async_copy(
                        rows[j], acc_sh.at[didx[prev]], sem_s[j]).wait()
                    if half == 0:
                        pl.when(k2 > 0)(drain)
                    else:
                        drain()
                    pltpu.async_copy(g_hbm.at[sidx[m]], rows[j], sem_g[j])

                for j in range(NSLOT):
                    m = half * NSLOT + j
                    ci = c0 + m
                    pltpu.make_async_copy(g_hbm.at[sidx[m]], rows[j],
                                          sem_g[j]).wait()
                    pltpu.async_copy(rows[j], acc_sh.at[didx[m]], sem_s[j],
                                     add=True)

                    @pl.when(ci + NSLOT < NCHUNK)
                    def _(ci=ci, m=m, j=j):
                        idx_load(ci + NSLOT, (m + NSLOT) % (2 * NSLOT),
                                 sem_i[j])

            return 0

        lax.fori_loop(0, NCHUNK // (2 * NSLOT), round_, 0)
        for j in range(NSLOT):
            pltpu.make_async_copy(
                rows[j], acc_sh.at[didx[NSLOT + j]], sem_s[j]).wait()

        plsc.subcore_barrier()
        done = 0
        while done < STRIPE:
            n = min(K, STRIPE - done)
            sl = pl.ds(s * STRIPE + done, n)
            pltpu.sync_copy(acc_sh.at[sl], rows[0].at[pl.ds(0, n)])
            pltpu.sync_copy(rows[0].at[pl.ds(0, n)], out_hbm.at[c, sl])
            done += n

    return k


def _sc_scalars(P, EPW):
    @functools.partial(
        pl.kernel,
        out_type=jax.ShapeDtypeStruct((NW, P), F32),
        mesh=_mesh(),
        compiler_params=pltpu.CompilerParams(needs_layout_passes=False),
        scratch_types=[
            pltpu.VMEM((EPW,), I32),
            pltpu.VMEM((EPW,), I32),
            pltpu.VMEM((P,), F32),
            pltpu.VMEM((P,), F32),
        ],
    )
    def k(zs_hbm, src_hbm, dst_hbm, out_hbm, sidx_v, didx_v, zs_v, acc_v):
        c = lax.axis_index("c")
        s = lax.axis_index("s")
        w = c * NS + s
        pltpu.sync_copy(zs_hbm, zs_v)
        pltpu.sync_copy(src_hbm.at[w], sidx_v)
        pltpu.sync_copy(dst_hbm.at[w], didx_v)

        def zero(i, _):
            acc_v[pl.ds(i * LANES, LANES)] = jnp.zeros((LANES,), F32)
            return 0

        lax.fori_loop(0, P // LANES, zero, 0)

        def body(j, _):
            sl = pl.ds(j * LANES, LANES)
            vals = plsc.load_gather(zs_v, [sidx_v[sl]])
            plsc.addupdate_scatter(acc_v, [didx_v[sl]], vals)
            return 0

        lax.fori_loop(0, EPW // LANES, body, 0)
        pltpu.sync_copy(acc_v, out_hbm.at[w])

    return k


def _tc_h(x_pad, W1, P, BR):
    def body(x_ref, w1_ref, h_ref):
        h_ref[...] = jnp.dot(x_ref[...], w1_ref[...],
                             preferred_element_type=F32)

    return pl.pallas_call(
        body,
        grid=(P // BR,),
        in_specs=[
            pl.BlockSpec((BR, 128), lambda i: (i, 0)),
            pl.BlockSpec((128, 128), lambda i: (0, 0)),
        ],
        out_specs=pl.BlockSpec((BR, 128), lambda i: (i, 0)),
        out_shape=jax.ShapeDtypeStruct((P, 128), F32),
    )(x_pad, W1)


def _tc_g(degT, h, P, BR):
    def body(deg_ref, h_ref, g_ref):
        deg = jnp.sum(deg_ref[...], axis=1, keepdims=True) + 1.0
        dis = lax.rsqrt(deg)
        g_ref[...] = dis * h_ref[...]

    return pl.pallas_call(
        body,
        grid=(P // BR,),
        in_specs=[
            pl.BlockSpec((BR, NW), lambda i: (i, 0)),
            pl.BlockSpec((BR, 128), lambda i: (i, 0)),
        ],
        out_specs=pl.BlockSpec((BR, 128), lambda i: (i, 0)),
        out_shape=jax.ShapeDtypeStruct((P, 128), F32),
    )(degT, h)


def _tc_zs(acc_part, g, degT, b1r, w2r, P, BR):
    def body(acc_ref, g_ref, deg_ref, b1_ref, w2_ref, zs_ref):
        acc = acc_ref[0] + acc_ref[1]
        deg = jnp.sum(deg_ref[...], axis=1, keepdims=True) + 1.0
        dis = lax.rsqrt(deg)
        h1 = jnp.maximum(dis * (acc + g_ref[...]) + b1_ref[...], 0.0)
        z = jnp.sum(h1 * w2_ref[...], axis=1, keepdims=True)
        zs_ref[...] = dis * z

    return pl.pallas_call(
        body,
        grid=(P // BR,),
        in_specs=[
            pl.BlockSpec((NC, BR, 128), lambda i: (0, i, 0)),
            pl.BlockSpec((BR, 128), lambda i: (i, 0)),
            pl.BlockSpec((BR, NW), lambda i: (i, 0)),
            pl.BlockSpec((1, 128), lambda i: (0, 0)),
            pl.BlockSpec((1, 128), lambda i: (0, 0)),
        ],
        out_specs=pl.BlockSpec((BR, 1), lambda i: (i, 0)),
        out_shape=jax.ShapeDtypeStruct((P, 1), F32),
    )(acc_part, g, degT, b1r, w2r)


def _tc_out(sacc2, zs2, deg2, b2r, P):
    R = P // 128

    def body(sacc_ref, zs_ref, deg_ref, b2_ref, out_ref):
        sacc = jnp.sum(sacc_ref[...], axis=0)
        deg = jnp.sum(deg_ref[...], axis=0) + 1.0
        dis = lax.rsqrt(deg)
        out_ref[...] = dis * (sacc + zs_ref[...]) + b2_ref[0, 0]

    return pl.pallas_call(
        body,
        out_shape=jax.ShapeDtypeStruct((R, 128), F32),
    )(sacc2, zs2, deg2, b2r)


def kernel(x, edge_index, W1, b1, W2, b2):
    N, D = x.shape
    H = W1.shape[1]
    E = edge_index.shape[1]

    P = -(-(N + 1) // 2048) * 2048
    EPW = -(-E // (NW * 2 * NSLOT * K)) * 2 * NSLOT * K
    EPAD = EPW * NW
    NCHUNK = EPW // K
    BR = 512

    x_pad = jnp.zeros((P, D), F32).at[:N].set(x)
    dum = N + jnp.arange(EPAD - E, dtype=I32) % (P - N)
    ei = jnp.concatenate(
        [edge_index.astype(I32), jnp.stack([dum, dum])], axis=1)
    src2 = ei[0].reshape(NW, EPW)
    dst2 = ei[1].reshape(NW, EPW)
    src3 = ei[0].reshape(NW, NCHUNK, K)
    dst3 = ei[1].reshape(NW, NCHUNK, K)

    h = _tc_h(x_pad, W1, P, BR)
    deg_part = _sc_degrees(P, EPW)(dst2)
    degT = deg_part.T

    g = _tc_g(degT, h, P, BR)
    acc_part = _sc_rows(P, NCHUNK)(g, src3, dst3)

    b1r = b1.reshape(1, H)
    w2r = W2.reshape(1, H)
    zs = _tc_zs(acc_part, g, degT, b1r, w2r, P, BR)

    sacc_part = _sc_scalars(P, EPW)(zs.reshape(P), src2, dst2)

    out2 = _tc_out(
        sacc_part.reshape(NW, P // 128, 128),
        zs.reshape(P // 128, 128),
        deg_part.reshape(NW, P // 128, 128),
        b2.reshape(1, 1),
        P,
    )
    return out2.reshape(-1)[:N]


# speedup vs baseline: 1.0008x; 1.0008x over previous
import functools

import jax
import jax.numpy as jnp
from jax import lax
from jax.experimental import pallas as pl
from jax.experimental.pallas import tpu as pltpu
from jax.experimental.pallas import tpu_sc as plsc

NC = 2
NS = 16
NW = NC * NS
LANES = 16
K = 128
NSLOT = 2

F32 = jnp.float32
I32 = jnp.int32


def _mesh():
    return plsc.VectorSubcoreMesh(core_axis_name="c", subcore_axis_name="s")


def _sc_degrees(P, EPW):
    @functools.partial(
        pl.kernel,
        out_type=jax.ShapeDtypeStruct((NW, P), F32),
        mesh=_mesh(),
        compiler_params=pltpu.CompilerParams(needs_layout_passes=False),
        scratch_types=[
            pltpu.VMEM((2, EPW), I32),
            pltpu.VMEM((P,), F32),
        ],
    )
    def k(ei_hbm, out_hbm, eix_v, acc_v):
        c = lax.axis_index("c")
        s = lax.axis_index("s")
        w = c * NS + s

        def zero(i, _):
            acc_v[pl.ds(i * LANES, LANES)] = jnp.zeros((LANES,), F32)
            return 0

        lax.fori_loop(0, P // LANES, zero, 0)
        pltpu.sync_copy(ei_hbm.at[:, w], eix_v)
        ones16 = jnp.ones((LANES,), F32)

        def body(j, _):
            idx = eix_v[1, pl.ds(j * LANES, LANES)]
            plsc.addupdate_scatter(acc_v, [idx], ones16)
            return 0

        lax.fori_loop(0, EPW // LANES, body, 0)
        pltpu.sync_copy(acc_v, out_hbm.at[w])

    return k


def _sc_rows(P, NCHUNK, EPW):
    STRIPE = P // NS

    @functools.partial(
        pl.kernel,
        out_type=jax.ShapeDtypeStruct((NC, P, 128), F32),
        mesh=_mesh(),
        compiler_params=pltpu.CompilerParams(needs_layout_passes=False),
        scratch_types=(
            [pltpu.VMEM((2, K), I32) for _ in range(2 * NSLOT)]
            + [pltpu.VMEM((K, 128), F32) for _ in range(NSLOT)]
            + [pltpu.VMEM_SHARED((P, 128), F32)]
            + [pltpu.SemaphoreType.DMA] * (3 * NSLOT)
        ),
    )
    def k(g_hbm, ei_hbm, out_hbm, *refs):
        eix = refs[0:2 * NSLOT]
        rows = refs[2 * NSLOT:3 * NSLOT]
        acc_sh = refs[3 * NSLOT]
        sem_i = refs[3 * NSLOT + 1:4 * NSLOT + 1]
        sem_g = refs[4 * NSLOT + 1:5 * NSLOT + 1]
        sem_s = refs[5 * NSLOT + 1:]

        c = lax.axis_index("c")
        s = lax.axis_index("s")
        w = c * NS + s

        zero16 = jnp.zeros((LANES,), F32)

        def zrow(r, _):
            for j in range(128 // LANES):
                rows[0][r, pl.ds(j * LANES, LANES)] = zero16
            return 0

        lax.fori_loop(0, K, zrow, 0)
        done = 0
        while done < STRIPE:
            n = min(K, STRIPE - done)
            pltpu.sync_copy(rows[0].at[pl.ds(0, n)],
                            acc_sh.at[pl.ds(s * STRIPE + done, n)])
            done += n
        plsc.subcore_barrier()

        def idx_load(ci, ring, sem):
            pltpu.async_copy(ei_hbm.at[:, w, ci], eix[ring], sem)

        def idx_wait(ci, ring, sem):
            pltpu.make_async_copy(ei_hbm.at[:, w, ci], eix[ring], sem).wait()

        for j in range(NSLOT):
            idx_load(j, j, sem_i[j])

        def round_(k2, _):
            c0 = 2 * NSLOT * k2
            for half in range(2):
                for j in range(NSLOT):
                    m = half * NSLOT + j
                    ci = c0 + m
                    idx_wait(ci, m, sem_i[j])
                    prev = (m + NSLOT) % (2 * NSLOT)
                    drain = lambda j=j, prev=prev: pltpu.make_async_copy(
                        rows[j], acc_sh.at[eix[prev].at[1]], sem_s[j]).wait()
                    if half == 0:
                        pl.when(k2 > 0)(drain)
                    else:
                        drain()
                    pltpu.async_copy(g_hbm.at[eix[m].at[0]], rows[j],
                                     sem_g[j])

                for j in range(NSLOT):
                    m = half * NSLOT + j
                    ci = c0 + m
                    pltpu.make_async_copy(g_hbm.at[eix[m].at[0]], rows[j],
                                          sem_g[j]).wait()
                    pltpu.async_copy(rows[j], acc_sh.at[eix[m].at[1]],
                                     sem_s[j], add=True)

                    @pl.when(ci + NSLOT < NCHUNK)
                    def _(ci=ci, m=m, j=j):
                        idx_load(ci + NSLOT, (m + NSLOT) % (2 * NSLOT),
                                 sem_i[j])

            return 0

        lax.fori_loop(0, NCHUNK // (2 * NSLOT), round_, 0)
        for j in range(NSLOT):
            pltpu.make_async_copy(
                rows[j], acc_sh.at[eix[NSLOT + j].at[1]], sem_s[j]).wait()

        plsc.subcore_barrier()
        done = 0
        while done < STRIPE:
            n = min(K, STRIPE - done)
            sl = pl.ds(s * STRIPE + done, n)
            pltpu.sync_copy(acc_sh.at[sl], rows[0].at[pl.ds(0, n)])
            pltpu.sync_copy(rows[0].at[pl.ds(0, n)], out_hbm.at[c, sl])
            done += n

    return k


def _sc_scalars(P, EPW):
    @functools.partial(
        pl.kernel,
        out_type=jax.ShapeDtypeStruct((NW, P), F32),
        mesh=_mesh(),
        compiler_params=pltpu.CompilerParams(needs_layout_passes=False),
        scratch_types=[
            pltpu.VMEM((2, EPW), I32),
            pltpu.VMEM((P,), F32),
            pltpu.VMEM((P,), F32),
        ],
    )
    def k(zs_hbm, ei_hbm, out_hbm, eix_v, zs_v, acc_v):
        c = lax.axis_index("c")
        s = lax.axis_index("s")
        w = c * NS + s
        pltpu.sync_copy(zs_hbm, zs_v)
        pltpu.sync_copy(ei_hbm.at[:, w], eix_v)

        def zero(i, _):
            acc_v[pl.ds(i * LANES, LANES)] = jnp.zeros((LANES,), F32)
            return 0

        lax.fori_loop(0, P // LANES, zero, 0)

        def body(j, _):
            sl = pl.ds(j * LANES, LANES)
            vals = plsc.load_gather(zs_v, [eix_v[0, sl]])
            plsc.addupdate_scatter(acc_v, [eix_v[1, sl]], vals)
            return 0

        lax.fori_loop(0, EPW // LANES, body, 0)
        pltpu.sync_copy(acc_v, out_hbm.at[w])

    return k


def _tc_g(degT, x_pad, W1, P, BR):
    def body(deg_ref, x_ref, w1_ref, g_ref):
        deg = jnp.sum(deg_ref[...], axis=1, keepdims=True) + 1.0
        dis = lax.rsqrt(deg)
        h = jnp.dot(x_ref[...], w1_ref[...], preferred_element_type=F32)
        g_ref[...] = dis * h

    return pl.pallas_call(
        body,
        grid=(P // BR,),
        in_specs=[
            pl.BlockSpec((BR, NW), lambda i: (i, 0)),
            pl.BlockSpec((BR, 128), lambda i: (i, 0)),
            pl.BlockSpec((128, 128), lambda i: (0, 0)),
        ],
        out_specs=pl.BlockSpec((BR, 128), lambda i: (i, 0)),
        out_shape=jax.ShapeDtypeStruct((P, 128), F32),
    )(degT, x_pad, W1)


def _tc_zs(acc_part, g, degT, b1r, w2r, P, BR):
    def body(acc_ref, g_ref, deg_ref, b1_ref, w2_ref, zs_ref):
        acc = acc_ref[0] + acc_ref[1]
        deg = jnp.sum(deg_ref[...], axis=1, keepdims=True) + 1.0
        dis = lax.rsqrt(deg)
        h1 = jnp.maximum(dis * (acc + g_ref[...]) + b1_ref[...], 0.0)
        z = jnp.sum(h1 * w2_ref[...], axis=1, keepdims=True)
        zs_ref[...] = dis * z

    return pl.pallas_call(
        body,
        grid=(P // BR,),
        in_specs=[
            pl.BlockSpec((NC, BR, 128), lambda i: (0, i, 0)),
            pl.BlockSpec((BR, 128), lambda i: (i, 0)),
            pl.BlockSpec((BR, NW), lambda i: (i, 0)),
            pl.BlockSpec((1, 128), lambda i: (0, 0)),
            pl.BlockSpec((1, 128), lambda i: (0, 0)),
        ],
        out_specs=pl.BlockSpec((BR, 1), lambda i: (i, 0)),
        out_shape=jax.ShapeDtypeStruct((P, 1), F32),
    )(acc_part, g, degT, b1r, w2r)


def _tc_out(sacc2, zs2, deg2, b2r, P):
    R = P // 128

    def body(sacc_ref, zs_ref, deg_ref, b2_ref, out_ref):
        sacc = jnp.sum(sacc_ref[...], axis=0)
        deg = jnp.sum(deg_ref[...], axis=0) + 1.0
        dis = lax.rsqrt(deg)
        out_ref[...] = dis * (sacc + zs_ref[...]) + b2_ref[0, 0]

    return pl.pallas_call(
        body,
        out_shape=jax.ShapeDtypeStruct((R, 128), F32),
    )(sacc2, zs2, deg2, b2r)


def kernel(x, edge_index, W1, b1, W2, b2):
    N, D = x.shape
    H = W1.shape[1]
    E = edge_index.shape[1]

    P = -(-(N + 1) // 2048) * 2048
    EPW = -(-E // (NW * 2 * NSLOT * K)) * 2 * NSLOT * K
    EPAD = EPW * NW
    NCHUNK = EPW // K
    BR = 512

    x_pad = jnp.zeros((P, D), F32).at[:N].set(x)
    dum = N + jnp.arange(EPAD - E, dtype=I32) % (P - N)
    ei = jnp.concatenate(
        [edge_index.astype(I32), jnp.stack([dum, dum])], axis=1)

    ei2 = ei.reshape(2, NW, EPW)
    ei4 = ei.reshape(2, NW, NCHUNK, K)

    deg_part = _sc_degrees(P, EPW)(ei2)
    degT = deg_part.T

    g = _tc_g(degT, x_pad, W1, P, BR)
    acc_part = _sc_rows(P, NCHUNK, EPW)(g, ei4)

    b1r = b1.reshape(1, H)
    w2r = W2.reshape(1, H)
    zs = _tc_zs(acc_part, g, degT, b1r, w2r, P, BR)

    sacc_part = _sc_scalars(P, EPW)(zs.reshape(P), ei2)

    out2 = _tc_out(
        sacc_part.reshape(NW, P // 128, 128),
        zs.reshape(P // 128, 128),
        deg_part.reshape(NW, P // 128, 128),
        b2.reshape(1, 1),
        P,
    )
    return out2.reshape(-1)[:N]


# speedup vs baseline: 1.1182x; 1.1173x over previous
import functools

import jax
import jax.numpy as jnp
from jax import lax
from jax.experimental import pallas as pl
from jax.experimental.pallas import tpu as pltpu
from jax.experimental.pallas import tpu_sc as plsc

NC = 2
NS = 16
NW = NC * NS
LANES = 16
K = 112
NSLOT = 3

F32 = jnp.float32
I32 = jnp.int32


def _mesh():
    return plsc.VectorSubcoreMesh(core_axis_name="c", subcore_axis_name="s")


def _sc_degrees(P, EPW):
    @functools.partial(
        pl.kernel,
        out_type=jax.ShapeDtypeStruct((NW, P), F32),
        mesh=_mesh(),
        compiler_params=pltpu.CompilerParams(needs_layout_passes=False),
        scratch_types=[
            pltpu.VMEM((2, EPW), I32),
            pltpu.VMEM((P,), F32),
        ],
    )
    def k(ei_hbm, out_hbm, eix_v, acc_v):
        c = lax.axis_index("c")
        s = lax.axis_index("s")
        w = c * NS + s

        def zero(i, _):
            acc_v[pl.ds(i * LANES, LANES)] = jnp.zeros((LANES,), F32)
            return 0

        lax.fori_loop(0, P // LANES, zero, 0)
        pltpu.sync_copy(ei_hbm.at[:, w], eix_v)
        ones16 = jnp.ones((LANES,), F32)

        def body(j, _):
            idx = eix_v[1, pl.ds(j * LANES, LANES)]
            plsc.addupdate_scatter(acc_v, [idx], ones16)
            return 0

        lax.fori_loop(0, EPW // LANES, body, 0)
        pltpu.sync_copy(acc_v, out_hbm.at[w])

    return k


def _sc_rows(P, NCHUNK, EPW):
    STRIPE = P // NS

    @functools.partial(
        pl.kernel,
        out_type=jax.ShapeDtypeStruct((NC, P, 128), F32),
        mesh=_mesh(),
        compiler_params=pltpu.CompilerParams(needs_layout_passes=False),
        scratch_types=(
            [pltpu.VMEM((2, K), I32) for _ in range(2 * NSLOT)]
            + [pltpu.VMEM((K, 128), F32) for _ in range(NSLOT)]
            + [pltpu.VMEM_SHARED((P, 128), F32)]
            + [pltpu.SemaphoreType.DMA] * (3 * NSLOT)
        ),
    )
    def k(g_hbm, ei_hbm, out_hbm, *refs):
        eix = refs[0:2 * NSLOT]
        rows = refs[2 * NSLOT:3 * NSLOT]
        acc_sh = refs[3 * NSLOT]
        sem_i = refs[3 * NSLOT + 1:4 * NSLOT + 1]
        sem_g = refs[4 * NSLOT + 1:5 * NSLOT + 1]
        sem_s = refs[5 * NSLOT + 1:]

        c = lax.axis_index("c")
        s = lax.axis_index("s")
        w = c * NS + s

        zero16 = jnp.zeros((LANES,), F32)

        def zrow(r, _):
            for j in range(128 // LANES):
                rows[0][r, pl.ds(j * LANES, LANES)] = zero16
            return 0

        lax.fori_loop(0, K, zrow, 0)
        done = 0
        while done < STRIPE:
            n = min(K, STRIPE - done)
            pltpu.sync_copy(rows[0].at[pl.ds(0, n)],
                            acc_sh.at[pl.ds(s * STRIPE + done, n)])
            done += n
        plsc.subcore_barrier()

        def idx_load(ci, ring, sem):
            pltpu.async_copy(ei_hbm.at[:, w, ci], eix[ring], sem)

        def idx_wait(ci, ring, sem):
            pltpu.make_async_copy(ei_hbm.at[:, w, ci], eix[ring], sem).wait()

        for j in range(NSLOT):
            idx_load(j, j, sem_i[j])

        def round_(k2, _):
            c0 = 2 * NSLOT * k2
            for half in range(2):
                for j in range(NSLOT):
                    m = half * NSLOT + j
                    ci = c0 + m
                    idx_wait(ci, m, sem_i[j])
                    prev = (m + NSLOT) % (2 * NSLOT)
                    drain = lambda j=j, prev=prev: pltpu.make_async_copy(
                        rows[j], acc_sh.at[eix[prev].at[1]], sem_s[j]).wait()
                    if half == 0:
                        pl.when(k2 > 0)(drain)
                    else:
                        drain()
                    pltpu.async_copy(g_hbm.at[eix[m].at[0]], rows[j],
                                     sem_g[j])

                for j in range(NSLOT):
                    m = half * NSLOT + j
                    ci = c0 + m
                    pltpu.make_async_copy(g_hbm.at[eix[m].at[0]], rows[j],
                                          sem_g[j]).wait()
                    pltpu.async_copy(rows[j], acc_sh.at[eix[m].at[1]],
                                     sem_s[j], add=True)

                    @pl.when(ci + NSLOT < NCHUNK)
                    def _(ci=ci, m=m, j=j):
                        idx_load(ci + NSLOT, (m + NSLOT) % (2 * NSLOT),
                                 sem_i[j])

            return 0

        lax.fori_loop(0, NCHUNK // (2 * NSLOT), round_, 0)
        for j in range(NSLOT):
            pltpu.make_async_copy(
                rows[j], acc_sh.at[eix[NSLOT + j].at[1]], sem_s[j]).wait()

        plsc.subcore_barrier()
        done = 0
        while done < STRIPE:
            n = min(K, STRIPE - done)
            sl = pl.ds(s * STRIPE + done, n)
            pltpu.sync_copy(acc_sh.at[sl], rows[0].at[pl.ds(0, n)])
            pltpu.sync_copy(rows[0].at[pl.ds(0, n)], out_hbm.at[c, sl])
            done += n

    return k


def _sc_scalars(P, EPW):
    @functools.partial(
        pl.kernel,
        out_type=jax.ShapeDtypeStruct((NW, P), F32),
        mesh=_mesh(),
        compiler_params=pltpu.CompilerParams(needs_layout_passes=False),
        scratch_types=[
            pltpu.VMEM((2, EPW), I32),
            pltpu.VMEM((P,), F32),
            pltpu.VMEM((P,), F32),
        ],
    )
    def k(zs_hbm, ei_hbm, out_hbm, eix_v, zs_v, acc_v):
        c = lax.axis_index("c")
        s = lax.axis_index("s")
        w = c * NS + s
        pltpu.sync_copy(zs_hbm, zs_v)
        pltpu.sync_copy(ei_hbm.at[:, w], eix_v)

        def zero(i, _):
            acc_v[pl.ds(i * LANES, LANES)] = jnp.zeros((LANES,), F32)
            return 0

        lax.fori_loop(0, P // LANES, zero, 0)

        def body(j, _):
            sl = pl.ds(j * LANES, LANES)
            vals = plsc.load_gather(zs_v, [eix_v[0, sl]])
            plsc.addupdate_scatter(acc_v, [eix_v[1, sl]], vals)
            return 0

        lax.fori_loop(0, EPW // LANES, body, 0)
        pltpu.sync_copy(acc_v, out_hbm.at[w])

    return k


def _tc_g(degT, x_pad, W1, P, BR):
    def body(deg_ref, x_ref, w1_ref, g_ref):
        deg = jnp.sum(deg_ref[...], axis=1, keepdims=True) + 1.0
        dis = lax.rsqrt(deg)
        h = jnp.dot(x_ref[...], w1_ref[...], preferred_element_type=F32)
        g_ref[...] = dis * h

    return pl.pallas_call(
        body,
        grid=(P // BR,),
        in_specs=[
            pl.BlockSpec((BR, NW), lambda i: (i, 0)),
            pl.BlockSpec((BR, 128), lambda i: (i, 0)),
            pl.BlockSpec((128, 128), lambda i: (0, 0)),
        ],
        out_specs=pl.BlockSpec((BR, 128), lambda i: (i, 0)),
        out_shape=jax.ShapeDtypeStruct((P, 128), F32),
    )(degT, x_pad, W1)


def _tc_zs(acc_part, g, degT, b1r, w2r, P, BR):
    def body(acc_ref, g_ref, deg_ref, b1_ref, w2_ref, zs_ref):
        acc = acc_ref[0] + acc_ref[1]
        deg = jnp.sum(deg_ref[...], axis=1, keepdims=True) + 1.0
        dis = lax.rsqrt(deg)
        h1 = jnp.maximum(dis * (acc + g_ref[...]) + b1_ref[...], 0.0)
        z = jnp.sum(h1 * w2_ref[...], axis=1, keepdims=True)
        zs_ref[...] = dis * z

    return pl.pallas_call(
        body,
        grid=(P // BR,),
        in_specs=[
            pl.BlockSpec((NC, BR, 128), lambda i: (0, i, 0)),
            pl.BlockSpec((BR, 128), lambda i: (i, 0)),
            pl.BlockSpec((BR, NW), lambda i: (i, 0)),
            pl.BlockSpec((1, 128), lambda i: (0, 0)),
            pl.BlockSpec((1, 128), lambda i: (0, 0)),
        ],
        out_specs=pl.BlockSpec((BR, 1), lambda i: (i, 0)),
        out_shape=jax.ShapeDtypeStruct((P, 1), F32),
    )(acc_part, g, degT, b1r, w2r)


def _tc_out(sacc2, zs2, deg2, b2r, P):
    R = P // 128

    def body(sacc_ref, zs_ref, deg_ref, b2_ref, out_ref):
        sacc = jnp.sum(sacc_ref[...], axis=0)
        deg = jnp.sum(deg_ref[...], axis=0) + 1.0
        dis = lax.rsqrt(deg)
        out_ref[...] = dis * (sacc + zs_ref[...]) + b2_ref[0, 0]

    return pl.pallas_call(
        body,
        out_shape=jax.ShapeDtypeStruct((R, 128), F32),
    )(sacc2, zs2, deg2, b2r)


def kernel(x, edge_index, W1, b1, W2, b2):
    N, D = x.shape
    H = W1.shape[1]
    E = edge_index.shape[1]

    P = -(-(N + 1) // 2048) * 2048
    EPW = -(-E // (NW * 2 * NSLOT * K)) * 2 * NSLOT * K
    EPAD = EPW * NW
    NCHUNK = EPW // K
    BR = 512

    x_pad = jnp.zeros((P, D), F32).at[:N].set(x)
    dum = N + jnp.arange(EPAD - E, dtype=I32) % (P - N)
    ei = jnp.concatenate(
        [edge_index.astype(I32), jnp.stack([dum, dum])], axis=1)

    ei2 = ei.reshape(2, NW, EPW)
    ei4 = ei.reshape(2, NW, NCHUNK, K)

    deg_part = _sc_degrees(P, EPW)(ei2)
    degT = deg_part.T

    g = _tc_g(degT, x_pad, W1, P, BR)
    acc_part = _sc_rows(P, NCHUNK, EPW)(g, ei4)

    b1r = b1.reshape(1, H)
    w2r = W2.reshape(1, H)
    zs = _tc_zs(acc_part, g, degT, b1r, w2r, P, BR)

    sacc_part = _sc_scalars(P, EPW)(zs.reshape(P), ei2)

    out2 = _tc_out(
        sacc_part.reshape(NW, P // 128, 128),
        zs.reshape(P // 128, 128),
        deg_part.reshape(NW, P // 128, 128),
        b2.reshape(1, 1),
        P,
    )
    return out2.reshape(-1)[:N]
